# baseline (speedup 1.0000x reference)
import jax
import jax.numpy as jnp
from jax import lax
from jax.experimental import pallas as pl
from jax.experimental.pallas import tpu as pltpu
from jax.experimental.pallas import tpu_sc as plsc

H = 128
LANES = 16
NEG = -1e30
NB = 6
NPB = 8448
NTILES = 32
CHUNK = 128


def _relu(v):
    return jnp.maximum(v, 0.0)


def _enc_nodes(x, W, b):
    N, K = x.shape
    BN = 2000

    def body(x_ref, w_ref, b_ref, o_ref):
        o_ref[...] = _relu(
            jnp.dot(x_ref[...], w_ref[...], preferred_element_type=jnp.float32)
            + b_ref[...])

    return pl.pallas_call(
        body,
        grid=(N // BN,),
        in_specs=[
            pl.BlockSpec((BN, K), lambda i: (i, 0)),
            pl.BlockSpec((K, H), lambda i: (0, 0)),
            pl.BlockSpec((1, H), lambda i: (0, 0)),
        ],
        out_specs=pl.BlockSpec((BN, H), lambda i: (i, 0)),
        out_shape=jax.ShapeDtypeStruct((N, H), jnp.float32),
    )(x, W, b.reshape(1, H))


def _pq(h, Ws, Wd):
    N = h.shape[0]
    BN = 2000

    def body(h_ref, ws_ref, wd_ref, op_ref, oq_ref):
        op_ref[...] = jnp.dot(h_ref[...], ws_ref[...],
                              preferred_element_type=jnp.float32)
        oq_ref[...] = jnp.dot(h_ref[...], wd_ref[...],
                              preferred_element_type=jnp.float32)

    return pl.pallas_call(
        body,
        grid=(N // BN,),
        in_specs=[
            pl.BlockSpec((BN, H), lambda i: (i, 0)),
            pl.BlockSpec((H, H), lambda i: (0, 0)),
            pl.BlockSpec((H, H), lambda i: (0, 0)),
        ],
        out_specs=[pl.BlockSpec((BN, H), lambda i: (i, 0))] * 2,
        out_shape=[jax.ShapeDtypeStruct((N, H), jnp.float32)] * 2,
    )(h, Ws, Wd)


def _enc_edges(ea_p, edge_W, edge_b):
    EP, KE = ea_p.shape
    BE = 2048

    def body(ea_ref, ew_ref, eb_ref, o_ref):
        o_ref[...] = _relu(
            jnp.dot(ea_ref[...], ew_ref[...], preferred_element_type=jnp.float32)
            + eb_ref[...])

    return pl.pallas_call(
        body,
        grid=(EP // BE,),
        in_specs=[
            pl.BlockSpec((BE, KE), lambda i: (i, 0)),
            pl.BlockSpec((KE, H), lambda i: (0, 0)),
            pl.BlockSpec((1, H), lambda i: (0, 0)),
        ],
        out_specs=pl.BlockSpec((BE, H), lambda i: (i, 0)),
        out_shape=jax.ShapeDtypeStruct((EP, H), jnp.float32),
    )(ea_p, edge_W, edge_b.reshape(1, H))


def _cs(es, perm2, We, mb, E):
    ES = es.shape[0]
    BE = 2048

    def body(e_ref, pm_ref, we_ref, mb_ref, o_ref):
        c = (jnp.dot(e_ref[...], we_ref[...], preferred_element_type=jnp.float32)
             + mb_ref[...])
        o_ref[...] = jnp.where(pm_ref[...] < E, c, NEG)

    return pl.pallas_call(
        body,
        grid=(ES // BE,),
        in_specs=[
            pl.BlockSpec((BE, H), lambda i: (i, 0)),
            pl.BlockSpec((BE, 1), lambda i: (i, 0)),
            pl.BlockSpec((H, H), lambda i: (0, 0)),
            pl.BlockSpec((1, H), lambda i: (0, 0)),
        ],
        out_specs=pl.BlockSpec((BE, H), lambda i: (i, 0)),
        out_shape=jax.ShapeDtypeStruct((ES, H), jnp.float32),
    )(es, perm2, We, mb.reshape(1, H))


def _update(h, aggp, U1, U2, b):
    N = h.shape[0]
    BN = 2000

    def body(h_ref, a_ref, u1_ref, u2_ref, b_ref, o_ref):
        agg = a_ref[0] + a_ref[1]
        acc = (jnp.dot(h_ref[...], u1_ref[...], preferred_element_type=jnp.float32)
               + jnp.dot(agg, u2_ref[...], preferred_element_type=jnp.float32)
               + b_ref[...])
        o_ref[...] = _relu(acc) + h_ref[...]

    return pl.pallas_call(
        body,
        grid=(N // BN,),
        in_specs=[
            pl.BlockSpec((BN, H), lambda i: (i, 0)),
            pl.BlockSpec((2, BN, H), lambda i: (0, i, 0)),
            pl.BlockSpec((H, H), lambda i: (0, 0)),
            pl.BlockSpec((H, H), lambda i: (0, 0)),
            pl.BlockSpec((1, H), lambda i: (0, 0)),
        ],
        out_specs=pl.BlockSpec((BN, H), lambda i: (i, 0)),
        out_shape=jax.ShapeDtypeStruct((N, H), jnp.float32),
    )(h, aggp, U1, U2, b.reshape(1, H))


def _head_ab(h, W1a, W1b, b1):
    N = h.shape[0]
    BN = 2000

    def body(h_ref, wa_ref, wb_ref, b_ref, oa_ref, ob_ref):
        oa_ref[...] = jnp.dot(h_ref[...], wa_ref[...],
                              preferred_element_type=jnp.float32) + b_ref[...]
        ob_ref[...] = jnp.dot(h_ref[...], wb_ref[...],
                              preferred_element_type=jnp.float32)

    return pl.pallas_call(
        body,
        grid=(N // BN,),
        in_specs=[
            pl.BlockSpec((BN, H), lambda i: (i, 0)),
            pl.BlockSpec((H, H), lambda i: (0, 0)),
            pl.BlockSpec((H, H), lambda i: (0, 0)),
            pl.BlockSpec((1, H), lambda i: (0, 0)),
        ],
        out_specs=[pl.BlockSpec((BN, H), lambda i: (i, 0))] * 2,
        out_shape=[jax.ShapeDtypeStruct((N, H), jnp.float32)] * 2,
    )(h, W1a, W1b, b1.reshape(1, H))


def _head_out(hidden, W2p, b2p):
    EP = hidden.shape[0]
    BE = 2048

    def body(h_ref, w_ref, b_ref, o_ref):
        o_ref[...] = jnp.dot(h_ref[...], w_ref[...],
                             preferred_element_type=jnp.float32) + b_ref[...]

    return pl.pallas_call(
        body,
        grid=(EP // BE,),
        in_specs=[
            pl.BlockSpec((BE, H), lambda i: (i, 0)),
            pl.BlockSpec((H, 8), lambda i: (0, 0)),
            pl.BlockSpec((1, 8), lambda i: (0, 0)),
        ],
        out_specs=pl.BlockSpec((BE, 8), lambda i: (i, 0)),
        out_shape=jax.ShapeDtypeStruct((EP, 8), jnp.float32),
    )(hidden, W2p, b2p.reshape(1, 8))


def _mesh():
    return plsc.VectorSubcoreMesh(core_axis_name="c", subcore_axis_name="s")


def _wid():
    return lax.axis_index("c") * 16 + lax.axis_index("s")


def _make_sc_hist(EP):
    EPT = EP // NTILES
    CH = EPT // CHUNK

    def body(dst_hbm, cnt_hbm, vdst, rbuf):
        w = _wid()
        ebase = w * EPT
        zi = jnp.zeros((LANES,), jnp.int32)
        one = jnp.ones((LANES,), jnp.int32)

        def zrow(i, carry):
            for j in range(8):
                rbuf[i, pl.ds(j * LANES, LANES)] = zi
            return carry

        lax.fori_loop(0, 8, zrow, 0)

        def chunk(t, accs):
            pltpu.sync_copy(dst_hbm.at[pl.ds(ebase + t * CHUNK, CHUNK)], vdst)
            accs = list(accs)
            for j in range(CHUNK // LANES):
                v = vdst[pl.ds(j * LANES, LANES)]
                for b in range(NB):
                    inb = (v >= b * NPB) & (v < (b + 1) * NPB)
                    accs[b] = accs[b] + jnp.where(inb, one, zi)
            return tuple(accs)

        accs = lax.fori_loop(0, CH, chunk, (zi,) * NB)
        li = lax.iota(jnp.int32, LANES)
        cv = zi
        for b in range(NB):
            cv = jnp.where(li == b, jnp.sum(accs[b]), cv)
        rbuf[0, pl.ds(0, LANES)] = cv
        pltpu.sync_copy(rbuf, cnt_hbm.at[w])

    return pl.kernel(
        body,
        out_type=jax.ShapeDtypeStruct((NTILES, 8, 128), jnp.int32),
        mesh=_mesh(),
        scratch_types=[
            pltpu.VMEM((CHUNK,), jnp.int32),
            pltpu.VMEM((8, 128), jnp.int32),
        ],
        compiler_params=pltpu.CompilerParams(needs_layout_passes=False),
    )


def _seg_starts(cnt_hbm, rbuf, w):
    def step(wq, carry):
        totals, mine = carry
        pltpu.sync_copy(cnt_hbm.at[wq], rbuf)
        v = rbuf[0, pl.ds(0, LANES)]
        new_tot, new_mine = [], []
        for b in range(NB):
            lw = ((v[b] + CHUNK - 1) // CHUNK) * CHUNK
            new_tot.append(totals[b] + lw)
            new_mine.append(mine[b] + jnp.where(wq < w, lw, 0))
        return tuple(new_tot), tuple(new_mine)

    z = (jnp.int32(0),) * NB
    totals, mine = lax.fori_loop(0, NTILES, step, (z, z))
    starts = []
    s = jnp.int32(0)
    for b in range(NB):
        starts.append(pl.multiple_of(s + mine[b], CHUNK))
        s = s + totals[b]
    pltpu.sync_copy(cnt_hbm.at[w], rbuf)
    v = rbuf[0, pl.ds(0, LANES)]
    nchunks = [(v[b] + CHUNK - 1) // CHUNK for b in range(NB)]
    return starts, nchunks


def _make_sc_bucket(EP, ES, E):
    EPT = EP // NTILES
    CH = EPT // CHUNK

    def body(src_hbm, dst_hbm, cnt_hbm, perm_hbm, srcs_hbm, dsts_hbm,
             rbuf, vsrc, vdst, pbuf, sbuf, dbuf):
        w = _wid()
        ebase = w * EPT
        starts, _ = _seg_starts(cnt_hbm, rbuf, w)
        li = lax.iota(jnp.int32, LANES)
        onev = jnp.ones((LANES,), jnp.int32)
        zerov = jnp.zeros((LANES,), jnp.int32)

        def flush_if_crossed(b, tb, cond):
            @pl.when(cond)
            def _():
                off = pl.multiple_of(lax.rem(tb - CHUNK, 256), CHUNK)
                dst_off = pl.multiple_of(starts[b] + tb - CHUNK, CHUNK)
                pltpu.sync_copy(pbuf.at[b, pl.ds(off, CHUNK)],
                                perm_hbm.at[pl.ds(dst_off, CHUNK)])
                pltpu.sync_copy(sbuf.at[b, pl.ds(off, CHUNK)],
                                srcs_hbm.at[pl.ds(dst_off, CHUNK)])
                pltpu.sync_copy(dbuf.at[b, pl.ds(off, CHUNK)],
                                dsts_hbm.at[pl.ds(dst_off, CHUNK)])

        def store16(v_o, v_s, v_d, tbs):
            bv = zerov
            for k in range(1, NB):
                bv = bv + jnp.where(v_d >= k * NPB, 1, 0)
            new = []
            for k in range(NB):
                mask = bv == k
                rank = plsc.cumsum(jnp.where(mask, onev, zerov), mask=mask)
                pos = tbs[k] + rank - 1
                slot = lax.rem(pos, 256)
                kvec = jnp.full((LANES,), k, jnp.int32)
                plsc.store_scatter(pbuf, [kvec, slot], v_o, mask=mask)
                plsc.store_scatter(sbuf, [kvec, slot], v_s, mask=mask)
                plsc.store_scatter(dbuf, [kvec, slot], v_d, mask=mask)
                tk = tbs[k] + jnp.sum(jnp.where(mask, onev, zerov))
                crossed = (tk // CHUNK) > (tbs[k] // CHUNK)
                flush_if_crossed(k, (tk // CHUNK) * CHUNK, crossed)
                new.append(tk)
            return tuple(new)

        def chunk(t, tbs):
            base = ebase + t * CHUNK
            pltpu.sync_copy(src_hbm.at[pl.ds(base, CHUNK)], vsrc)
            pltpu.sync_copy(dst_hbm.at[pl.ds(base, CHUNK)], vdst)
            for j in range(CHUNK // LANES):
                v_o = base + j * LANES + li
                v_s = vsrc[pl.ds(j * LANES, LANES)]
                v_d = vdst[pl.ds(j * LANES, LANES)]
                tbs = store16(v_o, v_s, v_d, tbs)
            return tbs

        tbs = lax.fori_loop(0, CH, chunk, (jnp.int32(0),) * NB)

        for b in range(NB):
            tb = tbs[b]
            end = ((tb + CHUNK - 1) // CHUNK) * CHUNK
            for j in range(CHUNK // LANES):
                pos = tb + j * LANES + li
                mask = pos < end
                slot = lax.rem(pos, 256)
                kvec = jnp.full((LANES,), b, jnp.int32)
                plsc.store_scatter(pbuf, [kvec, slot],
                                   jnp.full((LANES,), E, jnp.int32), mask=mask)
                plsc.store_scatter(sbuf, [kvec, slot], zerov, mask=mask)
                plsc.store_scatter(dbuf, [kvec, slot],
                                   jnp.full((LANES,), b * NPB, jnp.int32),
                                   mask=mask)
            flush_if_crossed(b, end, end > tb)

    return pl.kernel(
        body,
        out_type=[jax.ShapeDtypeStruct((ES,), jnp.int32)] * 3,
        mesh=_mesh(),
        scratch_types=[
            pltpu.VMEM((8, 128), jnp.int32),
            pltpu.VMEM((CHUNK,), jnp.int32),
            pltpu.VMEM((CHUNK,), jnp.int32),
            pltpu.VMEM((NB, 256), jnp.int32),
            pltpu.VMEM((NB, 256), jnp.int32),
            pltpu.VMEM((NB, 256), jnp.int32),
        ],
        compiler_params=pltpu.CompilerParams(needs_layout_passes=False),
    )


def _make_sc_esort(ES):
    IB = 8 * CHUNK

    def body(cnt_hbm, perm_hbm, e_hbm, out_hbm, rbuf, vperm, erow, sem):
        w = _wid()
        starts, nchunks = _seg_starts(cnt_hbm, rbuf, w)
        for b in range(NB):
            def chunk(t, carry, start=starts[b]):
                base = pl.multiple_of(start + t * CHUNK, CHUNK)

                @pl.when(lax.rem(t, 8) == 0)
                def _():
                    pltpu.sync_copy(perm_hbm.at[pl.ds(base, IB)], vperm)

                off = pl.multiple_of(lax.rem(t, 8) * CHUNK, CHUNK)
                pltpu.async_copy(e_hbm.at[vperm.at[pl.ds(off, CHUNK)]],
                                 erow, sem).wait()
                pltpu.sync_copy(erow, out_hbm.at[pl.ds(base, CHUNK)])
                return carry

            lax.fori_loop(0, nchunks[b], chunk, 0)

    return pl.kernel(
        body,
        out_type=jax.ShapeDtypeStruct((ES, H), jnp.float32),
        mesh=_mesh(),
        scratch_types=[
            pltpu.VMEM((8, 128), jnp.int32),
            pltpu.VMEM((IB,), jnp.int32),
            pltpu.VMEM((CHUNK, H), jnp.float32),
            pltpu.SemaphoreType.DMA,
        ],
        compiler_params=pltpu.CompilerParams(needs_layout_passes=False),
    )


def _make_sc_msg(N):
    RPT = NPB // 16
    ZR = 48
    MC = 64
    IB = 16 * MC

    def body(cnt_hbm, srcs_hbm, dsts_hbm, p_hbm, q_hbm, c_hbm,
             out_hbm, rbuf, vsrc, vdst, vdloc, prow, qrow, mrow,
             zbuf, table, sem0, sem1):
        cid = lax.axis_index("c")
        sid = lax.axis_index("s")
        w = cid * 16 + sid
        starts, nchunks = _seg_starts(cnt_hbm, rbuf, w)
        sems = (sem0, sem1)

        zv = jnp.zeros((LANES,), jnp.float32)

        def zrow(i, carry):
            for j in range(H // LANES):
                zbuf[i, pl.ds(j * LANES, LANES)] = zv
            return carry

        lax.fori_loop(0, ZR, zrow, 0)

        for b in range(NB):
            for j in range(RPT // ZR):
                pltpu.sync_copy(zbuf, table.at[pl.ds(sid * RPT + j * ZR, ZR)])
            plsc.subcore_barrier()

            start = starts[b]
            nmc = nchunks[b] * (CHUNK // MC)

            def issue(t, s, b=b, start=start):
                base = pl.multiple_of(start + t * MC, MC)

                @pl.when(lax.rem(t, 16) == 0)
                def _load_idx():
                    pltpu.sync_copy(srcs_hbm.at[pl.ds(base, IB)], vsrc)
                    pltpu.sync_copy(dsts_hbm.at[pl.ds(base, IB)], vdst)

                off = pl.multiple_of(lax.rem(t, 16) * MC, MC)
                for j in range(MC // LANES):
                    vdloc[s, pl.ds(j * LANES, LANES)] = (
                        vdst[pl.ds(off + j * LANES, LANES)] - b * NPB)
                vs = vsrc.at[pl.ds(off, MC)]
                vd = vdst.at[pl.ds(off, MC)]
                pltpu.async_copy(p_hbm.at[vs], prow.at[s], sems[s])
                pltpu.async_copy(q_hbm.at[vd], qrow.at[s], sems[s])
                pltpu.async_copy(c_hbm.at[pl.ds(base, MC)], mrow.at[s], sems[s])

            def drain(s):
                pltpu.make_async_copy(p_hbm.at[pl.ds(0, MC)], prow.at[s],
                                      sems[s]).wait()
                pltpu.make_async_copy(q_hbm.at[pl.ds(0, MC)], qrow.at[s],
                                      sems[s]).wait()
                pltpu.make_async_copy(c_hbm.at[pl.ds(0, MC)], mrow.at[s],
                                      sems[s]).wait()

            def compute(s):
                def comp(i, c2_):
                    for j in range(H // LANES):
                        v = (prow[s, i, pl.ds(j * LANES, LANES)]
                             + qrow[s, i, pl.ds(j * LANES, LANES)]
                             + mrow[s, i, pl.ds(j * LANES, LANES)])
                        mrow[s, i, pl.ds(j * LANES, LANES)] = jnp.maximum(v, 0.0)
                    return c2_

                lax.fori_loop(0, MC, comp, 0)
                pltpu.sync_copy(mrow.at[s], table.at[vdloc.at[s]], add=True)

            @pl.when(nmc > 0)
            def _prime():
                issue(0, 0)

            def pair(gp, carry, nmc=nmc):
                for s in range(2):
                    g = gp * 2 + s

                    @pl.when(g < nmc)
                    def _(g=g, s=s):
                        drain(s)

                        @pl.when(g + 1 < nmc)
                        def _():
                            issue(g + 1, 1 - s)

                        compute(s)
                return carry

            lax.fori_loop(0, (nmc + 1) // 2, pair, 0)
            plsc.subcore_barrier()
            pltpu.sync_copy(
                table.at[pl.ds(sid * RPT, RPT)],
                out_hbm.at[cid, pl.ds(b * NPB + sid * RPT, RPT)])
            plsc.subcore_barrier()

    return pl.kernel(
        body,
        out_type=jax.ShapeDtypeStruct((2, NB * NPB, H), jnp.float32),
        mesh=_mesh(),
        scratch_types=[
            pltpu.VMEM((8, 128), jnp.int32),
            pltpu.VMEM((IB,), jnp.int32),
            pltpu.VMEM((IB,), jnp.int32),
            pltpu.VMEM((2, MC), jnp.int32),
            pltpu.VMEM((2, MC, H), jnp.float32),
            pltpu.VMEM((2, MC, H), jnp.float32),
            pltpu.VMEM((2, MC, H), jnp.float32),
            pltpu.VMEM((48, H), jnp.float32),
            pltpu.VMEM_SHARED((NPB, H), jnp.float32),
            pltpu.SemaphoreType.DMA,
            pltpu.SemaphoreType.DMA,
        ],
        compiler_params=pltpu.CompilerParams(needs_layout_passes=False),
    )


def _make_sc_hidden(EP):
    EPT = EP // NTILES
    CH = EPT // CHUNK
    IB = 8 * CHUNK

    def body(src_hbm, dst_hbm, a_hbm, b_hbm, out_hbm,
             idx_s, idx_d, arow, brow, mrow, sem):
        w = _wid()
        ebase = w * EPT

        def chunk(t, carry):
            base = ebase + t * CHUNK

            @pl.when(lax.rem(t, 8) == 0)
            def _load_idx():
                pltpu.sync_copy(src_hbm.at[pl.ds(base, IB)], idx_s)
                pltpu.sync_copy(dst_hbm.at[pl.ds(base, IB)], idx_d)

            off = pl.multiple_of(lax.rem(t, 8) * CHUNK, CHUNK)
            ca = pltpu.async_copy(a_hbm.at[idx_s.at[pl.ds(off, CHUNK)]],
                                  arow, sem)
            cb = pltpu.async_copy(b_hbm.at[idx_d.at[pl.ds(off, CHUNK)]],
                                  brow, sem)
            ca.wait()
            cb.wait()

            def comp(i, c2_):
                for j in range(H // LANES):
                    v = arow[i, pl.ds(j * LANES, LANES)] + brow[i, pl.ds(j * LANES, LANES)]
                    mrow[i, pl.ds(j * LANES, LANES)] = jnp.maximum(v, 0.0)
                return c2_

            lax.fori_loop(0, CHUNK, comp, 0)
            pltpu.sync_copy(mrow, out_hbm.at[pl.ds(base, CHUNK)])
            return carry

        lax.fori_loop(0, CH, chunk, 0)

    return pl.kernel(
        body,
        out_type=jax.ShapeDtypeStruct((EP, H), jnp.float32),
        mesh=_mesh(),
        scratch_types=[
            pltpu.VMEM((IB,), jnp.int32),
            pltpu.VMEM((IB,), jnp.int32),
            pltpu.VMEM((CHUNK, H), jnp.float32),
            pltpu.VMEM((CHUNK, H), jnp.float32),
            pltpu.VMEM((CHUNK, H), jnp.float32),
            pltpu.SemaphoreType.DMA,
        ],
        compiler_params=pltpu.CompilerParams(needs_layout_passes=False),
    )


def kernel(x, edge_index, edge_attr, node_W, node_b, edge_W, edge_b,
           msg_W, msg_b, upd_W, upd_b, head_W1, head_b1, head_W2, head_b2):
    N = x.shape[0]
    E = edge_index.shape[1]
    L = msg_W.shape[0]
    EP = ((E + 4095) // 4096) * 4096
    ES = EP + NTILES * NB * CHUNK + 16 * CHUNK

    src = jnp.pad(edge_index[0], (0, EP - E + 8 * CHUNK))
    dst = jnp.pad(edge_index[1], (0, EP - E + 8 * CHUNK),
                  constant_values=N - 1)
    ea_p = jnp.pad(edge_attr, ((0, EP - E), (0, 0)))

    cnts = _make_sc_hist(EP)(dst)
    perm_s, src_s, dst_s = _make_sc_bucket(EP, ES, E)(src, dst, cnts)

    h = _enc_nodes(x, node_W, node_b)
    e_p = _enc_edges(ea_p, edge_W, edge_b)
    es = _make_sc_esort(ES)(cnts, perm_s, e_p)
    perm2 = perm_s.reshape(ES, 1)

    sc_msg = _make_sc_msg(N)
    for l in range(L):
        C = _cs(es, perm2, msg_W[l, 2 * H:], msg_b[l], E)
        P, Q = _pq(h, msg_W[l, :H], msg_W[l, H:2 * H])
        aggp = sc_msg(cnts, src_s, dst_s, P, Q, C)
        h = _update(h, aggp, upd_W[l, :H], upd_W[l, H:], upd_b[l])

    A, Bt = _head_ab(h, head_W1[:H], head_W1[H:], head_b1)
    hidden = _make_sc_hidden(EP)(src, dst, A, Bt)

    W2p = jnp.zeros((H, 8), jnp.float32).at[:, :2].set(head_W2)
    b2p = jnp.zeros((8,), jnp.float32).at[:2].set(head_b2)
    flow8 = _head_out(hidden, W2p, b2p)
    return flow8[:E, :2], h


# speedup vs baseline: 1.1742x; 1.1742x over previous
import jax
import jax.numpy as jnp
from jax import lax
from jax.experimental import pallas as pl
from jax.experimental.pallas import tpu as pltpu
from jax.experimental.pallas import tpu_sc as plsc

H = 128
LANES = 16
NEG = -1e30
NB = 6
NPB = 8448
NTILES = 32
CHUNK = 128


def _relu(v):
    return jnp.maximum(v, 0.0)


def _enc_nodes(x, W, b):
    N, K = x.shape
    BN = 2000

    def body(x_ref, w_ref, b_ref, o_ref):
        o_ref[...] = _relu(
            jnp.dot(x_ref[...], w_ref[...], preferred_element_type=jnp.float32)
            + b_ref[...])

    return pl.pallas_call(
        body,
        grid=(N // BN,),
        in_specs=[
            pl.BlockSpec((BN, K), lambda i: (i, 0)),
            pl.BlockSpec((K, H), lambda i: (0, 0)),
            pl.BlockSpec((1, H), lambda i: (0, 0)),
        ],
        out_specs=pl.BlockSpec((BN, H), lambda i: (i, 0)),
        out_shape=jax.ShapeDtypeStruct((N, H), jnp.float32),
    )(x, W, b.reshape(1, H))


def _pq(h, Ws, Wd):
    N = h.shape[0]
    BN = 2000

    def body(h_ref, ws_ref, wd_ref, op_ref, oq_ref):
        op_ref[...] = jnp.dot(h_ref[...], ws_ref[...],
                              preferred_element_type=jnp.float32)
        oq_ref[...] = jnp.dot(h_ref[...], wd_ref[...],
                              preferred_element_type=jnp.float32)

    return pl.pallas_call(
        body,
        grid=(N // BN,),
        in_specs=[
            pl.BlockSpec((BN, H), lambda i: (i, 0)),
            pl.BlockSpec((H, H), lambda i: (0, 0)),
            pl.BlockSpec((H, H), lambda i: (0, 0)),
        ],
        out_specs=[pl.BlockSpec((BN, H), lambda i: (i, 0))] * 2,
        out_shape=[jax.ShapeDtypeStruct((N, H), jnp.float32)] * 2,
    )(h, Ws, Wd)


def _cfull(ea_p, edge_W, edge_b, We, mb, E):
    EP, KE = ea_p.shape
    BE = 2048

    def body(ea_ref, ew_ref, eb_ref, we_ref, mb_ref, o_ref):
        i = pl.program_id(0)
        e = _relu(
            jnp.dot(ea_ref[...], ew_ref[...], preferred_element_type=jnp.float32)
            + eb_ref[...])
        c = jnp.dot(e, we_ref[...], preferred_element_type=jnp.float32) + mb_ref[...]
        rows = i * BE + lax.broadcasted_iota(jnp.int32, (BE, H), 0)
        o_ref[...] = jnp.where(rows < E, c, NEG)

    return pl.pallas_call(
        body,
        grid=(EP // BE,),
        in_specs=[
            pl.BlockSpec((BE, KE), lambda i: (i, 0)),
            pl.BlockSpec((KE, H), lambda i: (0, 0)),
            pl.BlockSpec((1, H), lambda i: (0, 0)),
            pl.BlockSpec((H, H), lambda i: (0, 0)),
            pl.BlockSpec((1, H), lambda i: (0, 0)),
        ],
        out_specs=pl.BlockSpec((BE, H), lambda i: (i, 0)),
        out_shape=jax.ShapeDtypeStruct((EP, H), jnp.float32),
    )(ea_p, edge_W, edge_b.reshape(1, H), We, mb.reshape(1, H))


def _update(h, aggp, U1, U2, b):
    N = h.shape[0]
    BN = 2000

    def body(h_ref, a_ref, u1_ref, u2_ref, b_ref, o_ref):
        agg = a_ref[0] + a_ref[1]
        acc = (jnp.dot(h_ref[...], u1_ref[...], preferred_element_type=jnp.float32)
               + jnp.dot(agg, u2_ref[...], preferred_element_type=jnp.float32)
               + b_ref[...])
        o_ref[...] = _relu(acc) + h_ref[...]

    return pl.pallas_call(
        body,
        grid=(N // BN,),
        in_specs=[
            pl.BlockSpec((BN, H), lambda i: (i, 0)),
            pl.BlockSpec((2, BN, H), lambda i: (0, i, 0)),
            pl.BlockSpec((H, H), lambda i: (0, 0)),
            pl.BlockSpec((H, H), lambda i: (0, 0)),
            pl.BlockSpec((1, H), lambda i: (0, 0)),
        ],
        out_specs=pl.BlockSpec((BN, H), lambda i: (i, 0)),
        out_shape=jax.ShapeDtypeStruct((N, H), jnp.float32),
    )(h, aggp, U1, U2, b.reshape(1, H))


def _head_ab(h, W1a, W1b, b1):
    N = h.shape[0]
    BN = 2000

    def body(h_ref, wa_ref, wb_ref, b_ref, oa_ref, ob_ref):
        oa_ref[...] = jnp.dot(h_ref[...], wa_ref[...],
                              preferred_element_type=jnp.float32) + b_ref[...]
        ob_ref[...] = jnp.dot(h_ref[...], wb_ref[...],
                              preferred_element_type=jnp.float32)

    return pl.pallas_call(
        body,
        grid=(N // BN,),
        in_specs=[
            pl.BlockSpec((BN, H), lambda i: (i, 0)),
            pl.BlockSpec((H, H), lambda i: (0, 0)),
            pl.BlockSpec((H, H), lambda i: (0, 0)),
            pl.BlockSpec((1, H), lambda i: (0, 0)),
        ],
        out_specs=[pl.BlockSpec((BN, H), lambda i: (i, 0))] * 2,
        out_shape=[jax.ShapeDtypeStruct((N, H), jnp.float32)] * 2,
    )(h, W1a, W1b, b1.reshape(1, H))


def _head_out(hidden, W2p, b2p):
    EP = hidden.shape[0]
    BE = 2048

    def body(h_ref, w_ref, b_ref, o_ref):
        o_ref[...] = jnp.dot(h_ref[...], w_ref[...],
                             preferred_element_type=jnp.float32) + b_ref[...]

    return pl.pallas_call(
        body,
        grid=(EP // BE,),
        in_specs=[
            pl.BlockSpec((BE, H), lambda i: (i, 0)),
            pl.BlockSpec((H, 8), lambda i: (0, 0)),
            pl.BlockSpec((1, 8), lambda i: (0, 0)),
        ],
        out_specs=pl.BlockSpec((BE, 8), lambda i: (i, 0)),
        out_shape=jax.ShapeDtypeStruct((EP, 8), jnp.float32),
    )(hidden, W2p, b2p.reshape(1, 8))


def _mesh():
    return plsc.VectorSubcoreMesh(core_axis_name="c", subcore_axis_name="s")


def _wid():
    return lax.axis_index("c") * 16 + lax.axis_index("s")


def _make_sc_hist(EP):
    EPT = EP // NTILES
    CH = EPT // CHUNK

    def body(dst_hbm, cnt_hbm, vdst, rbuf):
        w = _wid()
        ebase = w * EPT
        zi = jnp.zeros((LANES,), jnp.int32)
        one = jnp.ones((LANES,), jnp.int32)

        def zrow(i, carry):
            for j in range(8):
                rbuf[i, pl.ds(j * LANES, LANES)] = zi
            return carry

        lax.fori_loop(0, 8, zrow, 0)

        def chunk(t, accs):
            pltpu.sync_copy(dst_hbm.at[pl.ds(ebase + t * CHUNK, CHUNK)], vdst)
            accs = list(accs)
            for j in range(CHUNK // LANES):
                v = vdst[pl.ds(j * LANES, LANES)]
                for b in range(NB):
                    inb = (v >= b * NPB) & (v < (b + 1) * NPB)
                    accs[b] = accs[b] + jnp.where(inb, one, zi)
            return tuple(accs)

        accs = lax.fori_loop(0, CH, chunk, (zi,) * NB)
        li = lax.iota(jnp.int32, LANES)
        cv = zi
        for b in range(NB):
            cv = jnp.where(li == b, jnp.sum(accs[b]), cv)
        rbuf[0, pl.ds(0, LANES)] = cv
        pltpu.sync_copy(rbuf, cnt_hbm.at[w])

    return pl.kernel(
        body,
        out_type=jax.ShapeDtypeStruct((NTILES, 8, 128), jnp.int32),
        mesh=_mesh(),
        scratch_types=[
            pltpu.VMEM((CHUNK,), jnp.int32),
            pltpu.VMEM((8, 128), jnp.int32),
        ],
        compiler_params=pltpu.CompilerParams(needs_layout_passes=False),
    )


def _seg_starts(cnt_hbm, rbuf, w):
    def step(wq, carry):
        totals, mine = carry
        pltpu.sync_copy(cnt_hbm.at[wq], rbuf)
        v = rbuf[0, pl.ds(0, LANES)]
        new_tot, new_mine = [], []
        for b in range(NB):
            lw = ((v[b] + CHUNK - 1) // CHUNK) * CHUNK
            new_tot.append(totals[b] + lw)
            new_mine.append(mine[b] + jnp.where(wq < w, lw, 0))
        return tuple(new_tot), tuple(new_mine)

    z = (jnp.int32(0),) * NB
    totals, mine = lax.fori_loop(0, NTILES, step, (z, z))
    starts = []
    s = jnp.int32(0)
    for b in range(NB):
        starts.append(pl.multiple_of(s + mine[b], CHUNK))
        s = s + totals[b]
    pltpu.sync_copy(cnt_hbm.at[w], rbuf)
    v = rbuf[0, pl.ds(0, LANES)]
    nchunks = [(v[b] + CHUNK - 1) // CHUNK for b in range(NB)]
    return starts, nchunks


def _make_sc_bucket(EP, ES, E):
    EPT = EP // NTILES
    CH = EPT // CHUNK

    def body(src_hbm, dst_hbm, cnt_hbm, perm_hbm, srcs_hbm, dsts_hbm,
             rbuf, vsrc, vdst, pbuf, sbuf, dbuf):
        w = _wid()
        ebase = w * EPT
        starts, _ = _seg_starts(cnt_hbm, rbuf, w)
        li = lax.iota(jnp.int32, LANES)
        onev = jnp.ones((LANES,), jnp.int32)
        zerov = jnp.zeros((LANES,), jnp.int32)

        def flush_if_crossed(b, tb, cond):
            @pl.when(cond)
            def _():
                off = pl.multiple_of(lax.rem(tb - CHUNK, 256), CHUNK)
                dst_off = pl.multiple_of(starts[b] + tb - CHUNK, CHUNK)
                pltpu.sync_copy(pbuf.at[b, pl.ds(off, CHUNK)],
                                perm_hbm.at[pl.ds(dst_off, CHUNK)])
                pltpu.sync_copy(sbuf.at[b, pl.ds(off, CHUNK)],
                                srcs_hbm.at[pl.ds(dst_off, CHUNK)])
                pltpu.sync_copy(dbuf.at[b, pl.ds(off, CHUNK)],
                                dsts_hbm.at[pl.ds(dst_off, CHUNK)])

        def store16(v_o, v_s, v_d, tbs):
            bv = zerov
            for k in range(1, NB):
                bv = bv + jnp.where(v_d >= k * NPB, 1, 0)
            new = []
            for k in range(NB):
                mask = bv == k
                rank = plsc.cumsum(jnp.where(mask, onev, zerov), mask=mask)
                pos = tbs[k] + rank - 1
                slot = lax.rem(pos, 256)
                kvec = jnp.full((LANES,), k, jnp.int32)
                plsc.store_scatter(pbuf, [kvec, slot], v_o, mask=mask)
                plsc.store_scatter(sbuf, [kvec, slot], v_s, mask=mask)
                plsc.store_scatter(dbuf, [kvec, slot], v_d, mask=mask)
                tk = tbs[k] + jnp.sum(jnp.where(mask, onev, zerov))
                crossed = (tk // CHUNK) > (tbs[k] // CHUNK)
                flush_if_crossed(k, (tk // CHUNK) * CHUNK, crossed)
                new.append(tk)
            return tuple(new)

        def chunk(t, tbs):
            base = ebase + t * CHUNK
            pltpu.sync_copy(src_hbm.at[pl.ds(base, CHUNK)], vsrc)
            pltpu.sync_copy(dst_hbm.at[pl.ds(base, CHUNK)], vdst)
            for j in range(CHUNK // LANES):
                v_o = base + j * LANES + li
                v_s = vsrc[pl.ds(j * LANES, LANES)]
                v_d = vdst[pl.ds(j * LANES, LANES)]
                tbs = store16(v_o, v_s, v_d, tbs)
            return tbs

        tbs = lax.fori_loop(0, CH, chunk, (jnp.int32(0),) * NB)

        for b in range(NB):
            tb = tbs[b]
            end = ((tb + CHUNK - 1) // CHUNK) * CHUNK
            for j in range(CHUNK // LANES):
                pos = tb + j * LANES + li
                mask = pos < end
                slot = lax.rem(pos, 256)
                kvec = jnp.full((LANES,), b, jnp.int32)
                plsc.store_scatter(pbuf, [kvec, slot],
                                   jnp.full((LANES,), E, jnp.int32), mask=mask)
                plsc.store_scatter(sbuf, [kvec, slot], zerov, mask=mask)
                plsc.store_scatter(dbuf, [kvec, slot],
                                   jnp.full((LANES,), b * NPB, jnp.int32),
                                   mask=mask)
            flush_if_crossed(b, end, end > tb)

    return pl.kernel(
        body,
        out_type=[jax.ShapeDtypeStruct((ES,), jnp.int32)] * 3,
        mesh=_mesh(),
        scratch_types=[
            pltpu.VMEM((8, 128), jnp.int32),
            pltpu.VMEM((CHUNK,), jnp.int32),
            pltpu.VMEM((CHUNK,), jnp.int32),
            pltpu.VMEM((NB, 256), jnp.int32),
            pltpu.VMEM((NB, 256), jnp.int32),
            pltpu.VMEM((NB, 256), jnp.int32),
        ],
        compiler_params=pltpu.CompilerParams(needs_layout_passes=False),
    )


def _make_sc_msg(N):
    RPT = NPB // 16
    ZR = 48
    MC = 64
    IB = 16 * MC

    def body(cnt_hbm, perm_hbm, srcs_hbm, dsts_hbm, p_hbm, q_hbm, c_hbm,
             out_hbm, rbuf, vperm, vsrc, vdst, vdloc, prow, qrow, mrow,
             zbuf, table, sem0, sem1):
        cid = lax.axis_index("c")
        sid = lax.axis_index("s")
        w = cid * 16 + sid
        starts, nchunks = _seg_starts(cnt_hbm, rbuf, w)
        sems = (sem0, sem1)

        zv = jnp.zeros((LANES,), jnp.float32)

        def zrow(i, carry):
            for j in range(H // LANES):
                zbuf[i, pl.ds(j * LANES, LANES)] = zv
            return carry

        lax.fori_loop(0, ZR, zrow, 0)

        for b in range(NB):
            for j in range(RPT // ZR):
                pltpu.sync_copy(zbuf, table.at[pl.ds(sid * RPT + j * ZR, ZR)])
            plsc.subcore_barrier()

            start = starts[b]
            nmc = nchunks[b] * (CHUNK // MC)

            def issue(t, s, b=b, start=start):
                base = pl.multiple_of(start + t * MC, MC)

                @pl.when(lax.rem(t, 16) == 0)
                def _load_idx():
                    pltpu.sync_copy(perm_hbm.at[pl.ds(base, IB)], vperm)
                    pltpu.sync_copy(srcs_hbm.at[pl.ds(base, IB)], vsrc)
                    pltpu.sync_copy(dsts_hbm.at[pl.ds(base, IB)], vdst)

                off = pl.multiple_of(lax.rem(t, 16) * MC, MC)
                for j in range(MC // LANES):
                    vdloc[s, pl.ds(j * LANES, LANES)] = (
                        vdst[pl.ds(off + j * LANES, LANES)] - b * NPB)
                vs = vsrc.at[pl.ds(off, MC)]
                vd = vdst.at[pl.ds(off, MC)]
                vp = vperm.at[pl.ds(off, MC)]
                pltpu.async_copy(p_hbm.at[vs], prow.at[s], sems[s])
                pltpu.async_copy(q_hbm.at[vd], qrow.at[s], sems[s])
                pltpu.async_copy(c_hbm.at[vp], mrow.at[s], sems[s])

            def drain(s):
                pltpu.make_async_copy(p_hbm.at[pl.ds(0, MC)], prow.at[s],
                                      sems[s]).wait()
                pltpu.make_async_copy(q_hbm.at[pl.ds(0, MC)], qrow.at[s],
                                      sems[s]).wait()
                pltpu.make_async_copy(c_hbm.at[pl.ds(0, MC)], mrow.at[s],
                                      sems[s]).wait()

            def compute(s):
                def comp(i, c2_):
                    for j in range(H // LANES):
                        v = (prow[s, i, pl.ds(j * LANES, LANES)]
                             + qrow[s, i, pl.ds(j * LANES, LANES)]
                             + mrow[s, i, pl.ds(j * LANES, LANES)])
                        mrow[s, i, pl.ds(j * LANES, LANES)] = jnp.maximum(v, 0.0)
                    return c2_

                lax.fori_loop(0, MC, comp, 0)
                pltpu.sync_copy(mrow.at[s], table.at[vdloc.at[s]], add=True)

            @pl.when(nmc > 0)
            def _prime():
                issue(0, 0)

            def pair(gp, carry, nmc=nmc):
                for s in range(2):
                    g = gp * 2 + s

                    @pl.when(g < nmc)
                    def _(g=g, s=s):
                        drain(s)

                        @pl.when(g + 1 < nmc)
                        def _():
                            issue(g + 1, 1 - s)

                        compute(s)
                return carry

            lax.fori_loop(0, (nmc + 1) // 2, pair, 0)
            plsc.subcore_barrier()
            pltpu.sync_copy(
                table.at[pl.ds(sid * RPT, RPT)],
                out_hbm.at[cid, pl.ds(b * NPB + sid * RPT, RPT)])
            plsc.subcore_barrier()

    return pl.kernel(
        body,
        out_type=jax.ShapeDtypeStruct((2, NB * NPB, H), jnp.float32),
        mesh=_mesh(),
        scratch_types=[
            pltpu.VMEM((8, 128), jnp.int32),
            pltpu.VMEM((IB,), jnp.int32),
            pltpu.VMEM((IB,), jnp.int32),
            pltpu.VMEM((IB,), jnp.int32),
            pltpu.VMEM((2, MC), jnp.int32),
            pltpu.VMEM((2, MC, H), jnp.float32),
            pltpu.VMEM((2, MC, H), jnp.float32),
            pltpu.VMEM((2, MC, H), jnp.float32),
            pltpu.VMEM((48, H), jnp.float32),
            pltpu.VMEM_SHARED((NPB, H), jnp.float32),
            pltpu.SemaphoreType.DMA,
            pltpu.SemaphoreType.DMA,
        ],
        compiler_params=pltpu.CompilerParams(needs_layout_passes=False),
    )


def _make_sc_hidden(EP):
    EPT = EP // NTILES
    CH = EPT // CHUNK
    IB = 8 * CHUNK

    def body(src_hbm, dst_hbm, a_hbm, b_hbm, out_hbm,
             idx_s, idx_d, arow, brow, mrow, sem):
        w = _wid()
        ebase = w * EPT

        def chunk(t, carry):
            base = ebase + t * CHUNK

            @pl.when(lax.rem(t, 8) == 0)
            def _load_idx():
                pltpu.sync_copy(src_hbm.at[pl.ds(base, IB)], idx_s)
                pltpu.sync_copy(dst_hbm.at[pl.ds(base, IB)], idx_d)

            off = pl.multiple_of(lax.rem(t, 8) * CHUNK, CHUNK)
            ca = pltpu.async_copy(a_hbm.at[idx_s.at[pl.ds(off, CHUNK)]],
                                  arow, sem)
            cb = pltpu.async_copy(b_hbm.at[idx_d.at[pl.ds(off, CHUNK)]],
                                  brow, sem)
            ca.wait()
            cb.wait()

            def comp(i, c2_):
                for j in range(H // LANES):
                    v = arow[i, pl.ds(j * LANES, LANES)] + brow[i, pl.ds(j * LANES, LANES)]
                    mrow[i, pl.ds(j * LANES, LANES)] = jnp.maximum(v, 0.0)
                return c2_

            lax.fori_loop(0, CHUNK, comp, 0)
            pltpu.sync_copy(mrow, out_hbm.at[pl.ds(base, CHUNK)])
            return carry

        lax.fori_loop(0, CH, chunk, 0)

    return pl.kernel(
        body,
        out_type=jax.ShapeDtypeStruct((EP, H), jnp.float32),
        mesh=_mesh(),
        scratch_types=[
            pltpu.VMEM((IB,), jnp.int32),
            pltpu.VMEM((IB,), jnp.int32),
            pltpu.VMEM((CHUNK, H), jnp.float32),
            pltpu.VMEM((CHUNK, H), jnp.float32),
            pltpu.VMEM((CHUNK, H), jnp.float32),
            pltpu.SemaphoreType.DMA,
        ],
        compiler_params=pltpu.CompilerParams(needs_layout_passes=False),
    )


def kernel(x, edge_index, edge_attr, node_W, node_b, edge_W, edge_b,
           msg_W, msg_b, upd_W, upd_b, head_W1, head_b1, head_W2, head_b2):
    N = x.shape[0]
    E = edge_index.shape[1]
    L = msg_W.shape[0]
    EP = ((E + 4095) // 4096) * 4096
    ES = EP + NTILES * NB * CHUNK + 8 * CHUNK

    src = jnp.pad(edge_index[0], (0, EP - E + 8 * CHUNK))
    dst = jnp.pad(edge_index[1], (0, EP - E + 8 * CHUNK),
                  constant_values=N - 1)
    ea_p = jnp.pad(edge_attr, ((0, EP - E), (0, 0)))

    cnts = _make_sc_hist(EP)(dst)
    perm_s, src_s, dst_s = _make_sc_bucket(EP, ES, E)(src, dst, cnts)

    h = _enc_nodes(x, node_W, node_b)

    sc_msg = _make_sc_msg(N)
    for l in range(L):
        C = _cfull(ea_p, edge_W, edge_b, msg_W[l, 2 * H:], msg_b[l], E)
        P, Q = _pq(h, msg_W[l, :H], msg_W[l, H:2 * H])
        aggp = sc_msg(cnts, perm_s, src_s, dst_s, P, Q, C)
        h = _update(h, aggp, upd_W[l, :H], upd_W[l, H:], upd_b[l])

    A, Bt = _head_ab(h, head_W1[:H], head_W1[H:], head_b1)
    hidden = _make_sc_hidden(EP)(src, dst, A, Bt)

    W2p = jnp.zeros((H, 8), jnp.float32).at[:, :2].set(head_W2)
    b2p = jnp.zeros((8,), jnp.float32).at[:2].set(head_b2)
    flow8 = _head_out(hidden, W2p, b2p)
    return flow8[:E, :2], h
